# baseline (speedup 1.0000x reference)
import functools

import jax
import jax.numpy as jnp
from jax import lax
from jax.experimental import pallas as pl
from jax.experimental.pallas import tpu as pltpu
from jax.experimental.pallas import tpu_sc as plsc

N_CORES = 2
N_SUBCORES = 16
STREAM = 128


def _make_seg_sum(n_nodes, stream, spt, nbuf, with_cnt):
    own = -(-(n_nodes + 8) // (N_SUBCORES * 128)) * 128
    n_rows_acc = own * N_SUBCORES
    ch = max(d for d in range(8, stream + 1, 8) if own % d == 0)
    n_chunks = own // ch
    last0 = (N_SUBCORES - 1) * own
    full_last = (n_nodes - last0) // ch
    tail = n_nodes - last0 - full_last * ch

    out_type = [
        jax.ShapeDtypeStruct((n_nodes, 128), jnp.float32),
        jax.ShapeDtypeStruct((n_nodes, 128), jnp.float32),
    ]
    scratch = (
        [pltpu.VMEM((2, stream), jnp.int32) for _ in range(nbuf)]
        + [pltpu.VMEM((stream, 128), jnp.float32) for _ in range(nbuf)]
        + [pltpu.VMEM_SHARED((n_rows_acc, 128), jnp.float32)]
        + [pltpu.SemaphoreType.DMA for _ in range(nbuf)]
    )
    if with_cnt:
        out_type.append(jax.ShapeDtypeStruct((n_nodes, 16), jnp.float32))
        out_type.append(jax.ShapeDtypeStruct((n_nodes, 16), jnp.float32))
        scratch.append(pltpu.VMEM((max(stream, ch), 16), jnp.float32))
        scratch.append(pltpu.VMEM_SHARED((n_rows_acc, 16), jnp.float32))

    mesh = plsc.VectorSubcoreMesh(core_axis_name="c", subcore_axis_name="s")

    def body(ylo, yhi, eidx, zfeat, zcnt, ones_h, *rest):
        if with_cnt:
            out_lo, out_hi, out_cnt0, out_cnt1 = rest[:4]
            rest = rest[4:]
            onesv, cacc = rest[2 * nbuf + 1 + nbuf:]
        else:
            out_lo, out_hi = rest[:2]
            rest = rest[2:]
        ebuf = list(rest[:nbuf])
        rows = list(rest[nbuf:2 * nbuf])
        acc = rest[2 * nbuf]
        gsem = list(rest[2 * nbuf + 1:2 * nbuf + 1 + nbuf])
        rows0 = rows[0]
        tabs = [ylo, yhi]
        c = lax.axis_index("c")
        s = lax.axis_index("s")
        r0 = s * own

        zst = rows0.at[pl.ds(0, ch)]
        pltpu.sync_copy(zfeat.at[pl.ds(0, ch)], zst)
        for m in range(n_chunks):
            pltpu.sync_copy(zst, acc.at[pl.ds(r0 + m * ch, ch)])

        if with_cnt:
            ov = max(stream, ch)
            pltpu.sync_copy(zcnt.at[pl.ds(0, ov)], onesv)
            for m in range(own // ch):
                pltpu.sync_copy(onesv.at[pl.ds(0, ch)],
                                cacc.at[pl.ds(r0 + m * ch, ch)])
            pltpu.sync_copy(ones_h.at[pl.ds(0, ov)], onesv)

        plsc.subcore_barrier()

        g0 = s * spt

        def start_gather(g, b):
            pltpu.sync_copy(eidx.at[g0 + g], ebuf[b])
            for cc in range(N_CORES):
                @pl.when(c == cc)
                def _():
                    pltpu.async_copy(tabs[cc].at[ebuf[b].at[0]], rows[b],
                                     gsem[b])

        def wait_gather(b):
            for cc in range(N_CORES):
                @pl.when(c == cc)
                def _():
                    pltpu.make_async_copy(tabs[cc].at[ebuf[b].at[0]], rows[b],
                                          gsem[b]).wait()

        for b in range(nbuf):
            start_gather(b, b)

        def iter_body(k, carry):
            for b in range(nbuf):
                g = k * nbuf + b
                wait_gather(b)
                pltpu.sync_copy(rows[b], acc.at[ebuf[b].at[1]], add=True)
                if with_cnt:
                    @pl.when(c == (b % 2))
                    def _():
                        pltpu.sync_copy(onesv.at[pl.ds(0, stream)],
                                        cacc.at[ebuf[b].at[1]], add=True)

                @pl.when(g + nbuf < spt)
                def _():
                    start_gather(g + nbuf, b)
            return carry

        lax.fori_loop(0, spt // nbuf, iter_body, 0)

        plsc.subcore_barrier()

        out_feat = [out_lo, out_hi]

        def wb_feat(base, nrows):
            rst = rows0.at[pl.ds(0, nrows)]
            pltpu.sync_copy(acc.at[pl.ds(base, nrows)], rst)
            for cc in range(N_CORES):
                @pl.when(c == cc)
                def _():
                    pltpu.sync_copy(rst, out_feat[cc].at[pl.ds(base, nrows)])

        def wb_cnt(base, nrows):
            cst = onesv.at[pl.ds(0, nrows)]
            out_cnts = [out_cnt0, out_cnt1]
            for cc in range(N_CORES):
                @pl.when(c == cc)
                def _():
                    pltpu.sync_copy(cacc.at[pl.ds(base, nrows)], cst)
                    pltpu.sync_copy(cst, out_cnts[cc].at[pl.ds(base, nrows)])

        @pl.when(s < N_SUBCORES - 1)
        def _():
            for m in range(n_chunks):
                wb_feat(r0 + m * ch, ch)
                if with_cnt:
                    wb_cnt(r0 + m * ch, ch)

        @pl.when(s == N_SUBCORES - 1)
        def _():
            for m in range(full_last):
                wb_feat(last0 + m * ch, ch)
                if with_cnt:
                    wb_cnt(last0 + m * ch, ch)
            if tail:
                wb_feat(last0 + full_last * ch, tail)
                if with_cnt:
                    wb_cnt(last0 + full_last * ch, tail)

    return pl.kernel(
        body, out_type=out_type, mesh=mesh, scratch_types=scratch,
        compiler_params=pltpu.CompilerParams(use_tc_tiling_on_sc=False),
        name="seg_sum_sc")


def _tc1_body(x_ref, wl_ref, wr_ref, ylo_ref, yhi_ref, z_ref):
    xb = x_ref[...]
    yl = jnp.dot(xb, wl_ref[...], preferred_element_type=jnp.float32)
    ylo_ref[...] = yl[:, :128]
    yhi_ref[...] = yl[:, 128:]
    z_ref[...] = jnp.dot(xb, wr_ref[...], preferred_element_type=jnp.float32)


def _tc2_body(slo_ref, shi_ref, cnta_ref, cntb_ref, z1_ref, b1_ref,
              w2l_ref, w2r_ref, ylo_ref, yhi_ref, z2_ref):
    cnt = cnta_ref[...][:, 0:1] + cntb_ref[...][:, 0:1]
    inv = 1.0 / jnp.maximum(cnt, 1.0)
    sfull = jnp.concatenate([slo_ref[...], shi_ref[...]], axis=1)
    h = jnp.maximum(sfull * inv + b1_ref[...] + z1_ref[...], 0.0)
    y2 = jnp.dot(h, w2l_ref[...], preferred_element_type=jnp.float32)
    ylo_ref[...] = y2[:, :128]
    yhi_ref[...] = y2[:, 128:]
    z2_ref[...] = jnp.dot(h, w2r_ref[...], preferred_element_type=jnp.float32)


def _tc3_body(slo_ref, shi_ref, cnta_ref, cntb_ref, z2_ref, b2_ref, out_ref):
    cnt = cnta_ref[...][:, 0:1] + cntb_ref[...][:, 0:1]
    inv = 1.0 / jnp.maximum(cnt, 1.0)
    sfull = jnp.concatenate([slo_ref[...], shi_ref[...]], axis=1)
    out_ref[...] = sfull * inv + b2_ref[...] + z2_ref[...]


def _row_spec(nb, w):
    return pl.BlockSpec((nb, w), lambda i: (i, 0))


def _full_spec(shape):
    return pl.BlockSpec(shape, lambda i: tuple(0 for _ in shape))


def kernel(x, edge_index, W1l, b1, W1r, W2l, b2, W2r):
    n, d_in = x.shape
    h_dim = W1l.shape[1]
    e = edge_index.shape[1]

    stream1, stream2 = 88, 128
    nbuf1, nbuf2 = 2, 2

    def _spt(st, nb):
        q = -(-e // (st * N_SUBCORES))
        return -(-q // nb) * nb

    spt1, spt2 = _spt(stream1, nbuf1), _spt(stream2, nbuf2)

    def _mk_eidx(st, spt):
        tot = spt * N_SUBCORES * st
        sp = jnp.concatenate(
            [edge_index[0], jnp.zeros((tot - e,), jnp.int32)])
        dp = jnp.concatenate(
            [edge_index[1], jnp.full((tot - e,), n, jnp.int32)])
        return jnp.stack(
            [sp.reshape(-1, st), dp.reshape(-1, st)], axis=1)

    eidx1 = _mk_eidx(stream1, spt1)
    eidx2 = _mk_eidx(stream2, spt2)

    zfeat = jnp.zeros((STREAM, 128), jnp.float32)
    zcnt = jnp.zeros((STREAM, 16), jnp.float32)
    ones_h = jnp.ones((STREAM, 16), jnp.float32)

    nb = 1000
    grid = (n // nb,)

    y1lo, y1hi, z1 = pl.pallas_call(
        _tc1_body,
        grid=grid,
        in_specs=[_row_spec(nb, d_in), _full_spec((d_in, h_dim)),
                  _full_spec((d_in, h_dim))],
        out_specs=[_row_spec(nb, 128), _row_spec(nb, 128),
                   _row_spec(nb, h_dim)],
        out_shape=[jax.ShapeDtypeStruct((n, 128), jnp.float32),
                   jax.ShapeDtypeStruct((n, 128), jnp.float32),
                   jax.ShapeDtypeStruct((n, h_dim), jnp.float32)],
    )(x, W1l, W1r)

    seg1 = _make_seg_sum(n, stream1, spt1, nbuf1, with_cnt=True)
    s1lo, s1hi, cnt16a, cnt16b = seg1(y1lo, y1hi, eidx1, zfeat, zcnt, ones_h)

    y2lo, y2hi, z2 = pl.pallas_call(
        _tc2_body,
        grid=grid,
        in_specs=[_row_spec(nb, 128), _row_spec(nb, 128), _row_spec(nb, 16),
                  _row_spec(nb, 16), _row_spec(nb, h_dim),
                  _full_spec((1, h_dim)),
                  _full_spec((h_dim, h_dim)), _full_spec((h_dim, h_dim))],
        out_specs=[_row_spec(nb, 128), _row_spec(nb, 128),
                   _row_spec(nb, h_dim)],
        out_shape=[jax.ShapeDtypeStruct((n, 128), jnp.float32),
                   jax.ShapeDtypeStruct((n, 128), jnp.float32),
                   jax.ShapeDtypeStruct((n, h_dim), jnp.float32)],
    )(s1lo, s1hi, cnt16a, cnt16b, z1, b1.reshape(1, -1), W2l, W2r)

    seg2 = _make_seg_sum(n, stream2, spt2, nbuf2, with_cnt=False)
    s2lo, s2hi = seg2(y2lo, y2hi, eidx2, zfeat, zcnt, ones_h)

    out = pl.pallas_call(
        _tc3_body,
        grid=grid,
        in_specs=[_row_spec(nb, 128), _row_spec(nb, 128), _row_spec(nb, 16),
                  _row_spec(nb, 16), _row_spec(nb, h_dim),
                  _full_spec((1, h_dim))],
        out_specs=_row_spec(nb, h_dim),
        out_shape=jax.ShapeDtypeStruct((n, h_dim), jnp.float32),
    )(s2lo, s2hi, cnt16a, cnt16b, z2, b2.reshape(1, -1))

    return out


# speedup vs baseline: 1.0365x; 1.0365x over previous
import functools

import jax
import jax.numpy as jnp
from jax import lax
from jax.experimental import pallas as pl
from jax.experimental.pallas import tpu as pltpu
from jax.experimental.pallas import tpu_sc as plsc

N_CORES = 2
N_SUBCORES = 16
STREAM = 128


def _make_seg_sum(n_nodes, stream, spt, nbuf, with_cnt):
    own = -(-(n_nodes + 8) // (N_SUBCORES * 128)) * 128
    n_rows_acc = own * N_SUBCORES
    ch = max(d for d in range(8, stream + 1, 8) if own % d == 0)
    n_chunks = own // ch
    last0 = (N_SUBCORES - 1) * own
    full_last = (n_nodes - last0) // ch
    tail = n_nodes - last0 - full_last * ch

    out_type = [
        jax.ShapeDtypeStruct((n_nodes, 128), jnp.float32),
        jax.ShapeDtypeStruct((n_nodes, 128), jnp.float32),
    ]
    scratch = (
        [pltpu.VMEM((2, stream), jnp.int32) for _ in range(nbuf)]
        + [pltpu.VMEM((stream, 128), jnp.float32) for _ in range(nbuf)]
        + [pltpu.VMEM_SHARED((n_rows_acc, 128), jnp.float32)]
        + [pltpu.SemaphoreType.DMA for _ in range(nbuf)]
    )
    if with_cnt:
        out_type.append(jax.ShapeDtypeStruct((n_nodes, 16), jnp.float32))
        out_type.append(jax.ShapeDtypeStruct((n_nodes, 16), jnp.float32))
        scratch.append(pltpu.VMEM((max(stream, ch), 16), jnp.float32))
        scratch.append(pltpu.VMEM_SHARED((n_rows_acc, 16), jnp.float32))

    mesh = plsc.VectorSubcoreMesh(core_axis_name="c", subcore_axis_name="s")

    def body(ylo, yhi, eidx, zfeat, zcnt, ones_h, *rest):
        if with_cnt:
            out_lo, out_hi, out_cnt0, out_cnt1 = rest[:4]
            rest = rest[4:]
            onesv, cacc = rest[2 * nbuf + 1 + nbuf:]
        else:
            out_lo, out_hi = rest[:2]
            rest = rest[2:]
        ebuf = list(rest[:nbuf])
        rows = list(rest[nbuf:2 * nbuf])
        acc = rest[2 * nbuf]
        gsem = list(rest[2 * nbuf + 1:2 * nbuf + 1 + nbuf])
        rows0 = rows[0]
        tabs = [ylo, yhi]
        c = lax.axis_index("c")
        s = lax.axis_index("s")
        r0 = s * own

        zst = rows0.at[pl.ds(0, ch)]
        pltpu.sync_copy(zfeat.at[pl.ds(0, ch)], zst)
        for m in range(n_chunks):
            pltpu.sync_copy(zst, acc.at[pl.ds(r0 + m * ch, ch)])

        if with_cnt:
            ov = max(stream, ch)
            pltpu.sync_copy(zcnt.at[pl.ds(0, ov)], onesv)
            for m in range(own // ch):
                pltpu.sync_copy(onesv.at[pl.ds(0, ch)],
                                cacc.at[pl.ds(r0 + m * ch, ch)])
            pltpu.sync_copy(ones_h.at[pl.ds(0, ov)], onesv)

        plsc.subcore_barrier()

        g0 = s * spt

        def start_gather(g, b):
            pltpu.sync_copy(eidx.at[g0 + g], ebuf[b])
            for cc in range(N_CORES):
                @pl.when(c == cc)
                def _():
                    pltpu.async_copy(tabs[cc].at[ebuf[b].at[0]], rows[b],
                                     gsem[b])

        def wait_gather(b):
            for cc in range(N_CORES):
                @pl.when(c == cc)
                def _():
                    pltpu.make_async_copy(tabs[cc].at[ebuf[b].at[0]], rows[b],
                                          gsem[b]).wait()

        for b in range(nbuf):
            start_gather(b, b)

        def iter_body(k, carry):
            for b in range(nbuf):
                g = k * nbuf + b
                wait_gather(b)
                pltpu.sync_copy(rows[b], acc.at[ebuf[b].at[1]], add=True)
                if with_cnt:
                    @pl.when(c == (b % 2))
                    def _():
                        pltpu.sync_copy(onesv.at[pl.ds(0, stream)],
                                        cacc.at[ebuf[b].at[1]], add=True)

                @pl.when(g + nbuf < spt)
                def _():
                    start_gather(g + nbuf, b)
            return carry

        lax.fori_loop(0, spt // nbuf, iter_body, 0)

        plsc.subcore_barrier()

        out_feat = [out_lo, out_hi]

        def wb_feat(base, nrows):
            rst = rows0.at[pl.ds(0, nrows)]
            pltpu.sync_copy(acc.at[pl.ds(base, nrows)], rst)
            for cc in range(N_CORES):
                @pl.when(c == cc)
                def _():
                    pltpu.sync_copy(rst, out_feat[cc].at[pl.ds(base, nrows)])

        def wb_cnt(base, nrows):
            cst = onesv.at[pl.ds(0, nrows)]
            out_cnts = [out_cnt0, out_cnt1]
            for cc in range(N_CORES):
                @pl.when(c == cc)
                def _():
                    pltpu.sync_copy(cacc.at[pl.ds(base, nrows)], cst)
                    pltpu.sync_copy(cst, out_cnts[cc].at[pl.ds(base, nrows)])

        @pl.when(s < N_SUBCORES - 1)
        def _():
            for m in range(n_chunks):
                wb_feat(r0 + m * ch, ch)
                if with_cnt:
                    wb_cnt(r0 + m * ch, ch)

        @pl.when(s == N_SUBCORES - 1)
        def _():
            for m in range(full_last):
                wb_feat(last0 + m * ch, ch)
                if with_cnt:
                    wb_cnt(last0 + m * ch, ch)
            if tail:
                wb_feat(last0 + full_last * ch, tail)
                if with_cnt:
                    wb_cnt(last0 + full_last * ch, tail)

    return pl.kernel(
        body, out_type=out_type, mesh=mesh, scratch_types=scratch,
        compiler_params=pltpu.CompilerParams(use_tc_tiling_on_sc=False),
        name="seg_sum_sc")


def _tc1_body(x_ref, wl_ref, wr_ref, ylo_ref, yhi_ref, z_ref):
    xb = x_ref[...]
    yl = jnp.dot(xb, wl_ref[...], preferred_element_type=jnp.float32)
    ylo_ref[...] = yl[:, :128]
    yhi_ref[...] = yl[:, 128:]
    z_ref[...] = jnp.dot(xb, wr_ref[...], preferred_element_type=jnp.float32)


def _tc2_body(slo_ref, shi_ref, cnta_ref, cntb_ref, z1_ref, b1_ref,
              w2l_ref, w2r_ref, ylo_ref, yhi_ref, z2_ref):
    cnt = cnta_ref[...][:, 0:1] + cntb_ref[...][:, 0:1]
    inv = 1.0 / jnp.maximum(cnt, 1.0)
    sfull = jnp.concatenate([slo_ref[...], shi_ref[...]], axis=1)
    h = jnp.maximum(sfull * inv + b1_ref[...] + z1_ref[...], 0.0)
    y2 = jnp.dot(h, w2l_ref[...], preferred_element_type=jnp.float32)
    ylo_ref[...] = y2[:, :128]
    yhi_ref[...] = y2[:, 128:]
    z2_ref[...] = jnp.dot(h, w2r_ref[...], preferred_element_type=jnp.float32)


def _tc3_body(slo_ref, shi_ref, cnta_ref, cntb_ref, z2_ref, b2_ref, out_ref):
    cnt = cnta_ref[...][:, 0:1] + cntb_ref[...][:, 0:1]
    inv = 1.0 / jnp.maximum(cnt, 1.0)
    sfull = jnp.concatenate([slo_ref[...], shi_ref[...]], axis=1)
    out_ref[...] = sfull * inv + b2_ref[...] + z2_ref[...]


def _row_spec(nb, w):
    return pl.BlockSpec((nb, w), lambda i: (i, 0))


def _full_spec(shape):
    return pl.BlockSpec(shape, lambda i: tuple(0 for _ in shape))


def kernel(x, edge_index, W1l, b1, W1r, W2l, b2, W2r):
    n, d_in = x.shape
    h_dim = W1l.shape[1]
    e = edge_index.shape[1]

    stream1, stream2 = 80, 128
    nbuf1, nbuf2 = 2, 2

    def _spt(st, nb):
        q = -(-e // (st * N_SUBCORES))
        return -(-q // nb) * nb

    spt1, spt2 = _spt(stream1, nbuf1), _spt(stream2, nbuf2)

    def _mk_eidx(st, spt):
        tot = spt * N_SUBCORES * st
        sp = jnp.concatenate(
            [edge_index[0], jnp.zeros((tot - e,), jnp.int32)])
        dp = jnp.concatenate(
            [edge_index[1], jnp.full((tot - e,), n, jnp.int32)])
        return jnp.stack(
            [sp.reshape(-1, st), dp.reshape(-1, st)], axis=1)

    eidx1 = _mk_eidx(stream1, spt1)
    eidx2 = _mk_eidx(stream2, spt2)

    zfeat = jnp.zeros((STREAM, 128), jnp.float32)
    zcnt = jnp.zeros((STREAM, 16), jnp.float32)
    ones_h = jnp.ones((STREAM, 16), jnp.float32)

    nb = 1000
    grid = (n // nb,)

    y1lo, y1hi, z1 = pl.pallas_call(
        _tc1_body,
        grid=grid,
        in_specs=[_row_spec(nb, d_in), _full_spec((d_in, h_dim)),
                  _full_spec((d_in, h_dim))],
        out_specs=[_row_spec(nb, 128), _row_spec(nb, 128),
                   _row_spec(nb, h_dim)],
        out_shape=[jax.ShapeDtypeStruct((n, 128), jnp.float32),
                   jax.ShapeDtypeStruct((n, 128), jnp.float32),
                   jax.ShapeDtypeStruct((n, h_dim), jnp.float32)],
    )(x, W1l, W1r)

    seg1 = _make_seg_sum(n, stream1, spt1, nbuf1, with_cnt=True)
    s1lo, s1hi, cnt16a, cnt16b = seg1(y1lo, y1hi, eidx1, zfeat, zcnt, ones_h)

    y2lo, y2hi, z2 = pl.pallas_call(
        _tc2_body,
        grid=grid,
        in_specs=[_row_spec(nb, 128), _row_spec(nb, 128), _row_spec(nb, 16),
                  _row_spec(nb, 16), _row_spec(nb, h_dim),
                  _full_spec((1, h_dim)),
                  _full_spec((h_dim, h_dim)), _full_spec((h_dim, h_dim))],
        out_specs=[_row_spec(nb, 128), _row_spec(nb, 128),
                   _row_spec(nb, h_dim)],
        out_shape=[jax.ShapeDtypeStruct((n, 128), jnp.float32),
                   jax.ShapeDtypeStruct((n, 128), jnp.float32),
                   jax.ShapeDtypeStruct((n, h_dim), jnp.float32)],
    )(s1lo, s1hi, cnt16a, cnt16b, z1, b1.reshape(1, -1), W2l, W2r)

    seg2 = _make_seg_sum(n, stream2, spt2, nbuf2, with_cnt=False)
    s2lo, s2hi = seg2(y2lo, y2hi, eidx2, zfeat, zcnt, ones_h)

    out = pl.pallas_call(
        _tc3_body,
        grid=grid,
        in_specs=[_row_spec(nb, 128), _row_spec(nb, 128), _row_spec(nb, 16),
                  _row_spec(nb, 16), _row_spec(nb, h_dim),
                  _full_spec((1, h_dim))],
        out_specs=_row_spec(nb, h_dim),
        out_shape=jax.ShapeDtypeStruct((n, h_dim), jnp.float32),
    )(s2lo, s2hi, cnt16a, cnt16b, z2, b2.reshape(1, -1))

    return out
